# baseline (speedup 1.0000x reference)
import jax
import jax.numpy as jnp
from jax.experimental import pallas as pl
from jax.experimental.pallas import tpu as pltpu

_N = 10000
_F = 128
_BR1 = 320
_NB1 = -(-_N // _BR1)
_BR2 = 1024
_NB2 = -(-_N // _BR2)


def _layer1_body(xp_ref, adj_ref, W_ref, b_ref, h1_ref, adjc_ref, S_ref):
    i = pl.program_id(0)

    @pl.when(i == 0)
    def _compute_support():
        S_ref[...] = jnp.dot(xp_ref[...], W_ref[0],
                             preferred_element_type=jnp.float32)

    a = adj_ref[...]
    adjc_ref[...] = a.astype(jnp.float8_e4m3fn)
    acc = jnp.dot(a, S_ref[...], preferred_element_type=jnp.float32)
    h1_ref[...] = jnp.maximum(acc + b_ref[0], 0.0)


def _layer23_body(h1_ref, adjc_ref, W_ref, b_ref, out_ref,
                  Sc_ref, H_ref, sc_ref):
    l = pl.program_id(0)
    i = pl.program_id(1)

    def _quantize_support(prev, W):
        S = jnp.dot(prev, W, preferred_element_type=jnp.float32)
        s1 = jnp.maximum(jnp.max(jnp.abs(S)) / 448.0, 1e-30)
        Sh = (S / s1).astype(jnp.float8_e4m3fn)
        Rr = S - Sh.astype(jnp.float32) * s1
        s2 = jnp.maximum(jnp.max(jnp.abs(Rr)) / 448.0, 1e-30)
        Sc_ref[:, 0:_F] = Sh
        Sc_ref[:, _F:2 * _F] = (Rr / s2).astype(jnp.float8_e4m3fn)
        sc_ref[0] = s1
        sc_ref[1] = s2

    @pl.when(jnp.logical_and(l == 0, i == 0))
    def _support_l2():
        _quantize_support(h1_ref[...], W_ref[1])

    @pl.when(jnp.logical_and(l == 1, i == 0))
    def _support_l3():
        _quantize_support(H_ref[0:_N, :], W_ref[2])

    acc_w = jnp.dot(adjc_ref[...], Sc_ref[...],
                    preferred_element_type=jnp.float32)
    acc = (sc_ref[0] * acc_w[:, 0:_F]
           + sc_ref[1] * acc_w[:, _F:2 * _F]
           + b_ref[l + 1])
    h = jnp.where(l == 0, jnp.maximum(acc, 0.0), acc)
    H_ref[pl.ds(i * _BR2, _BR2), :] = h
    out_ref[...] = h[:, 0:2]


def kernel(x, adj, W1, b1, W2, b2, W3, b3):
    xp = jnp.zeros((_N, _F), jnp.float32).at[:, :2].set(x)
    Wp = (jnp.zeros((3, _F, _F), jnp.float32)
          .at[0, :2, :5].set(W1)
          .at[1, :5, :5].set(W2)
          .at[2, :5, :2].set(W3))
    bp = (jnp.zeros((3, 1, _F), jnp.float32)
          .at[0, 0, :5].set(b1)
          .at[1, 0, :5].set(b2)
          .at[2, 0, :2].set(b3))

    h1, adjc = pl.pallas_call(
        _layer1_body,
        grid=(_NB1,),
        in_specs=[
            pl.BlockSpec((_N, _F), lambda i: (0, 0)),
            pl.BlockSpec((_BR1, _N), lambda i: (i, 0)),
            pl.BlockSpec((3, _F, _F), lambda i: (0, 0, 0)),
            pl.BlockSpec((3, 1, _F), lambda i: (0, 0, 0)),
        ],
        out_specs=[
            pl.BlockSpec((_BR1, _F), lambda i: (i, 0)),
            pl.BlockSpec((_BR1, _N), lambda i: (i, 0)),
        ],
        out_shape=[
            jax.ShapeDtypeStruct((_N, _F), jnp.float32),
            jax.ShapeDtypeStruct((_N, _N), jnp.float8_e4m3fn),
        ],
        scratch_shapes=[pltpu.VMEM((_N, _F), jnp.float32)],
        compiler_params=pltpu.CompilerParams(
            dimension_semantics=("arbitrary",),
        ),
    )(xp, adj, Wp, bp)

    out = pl.pallas_call(
        _layer23_body,
        grid=(2, _NB2),
        in_specs=[
            pl.BlockSpec((_N, _F), lambda l, i: (0, 0)),
            pl.BlockSpec((_BR2, _N), lambda l, i: (i, 0)),
            pl.BlockSpec((3, _F, _F), lambda l, i: (0, 0, 0)),
            pl.BlockSpec((3, 1, _F), lambda l, i: (0, 0, 0)),
        ],
        out_specs=pl.BlockSpec((_BR2, 2), lambda l, i: (i, 0)),
        out_shape=jax.ShapeDtypeStruct((_N, 2), jnp.float32),
        scratch_shapes=[
            pltpu.VMEM((_N, 2 * _F), jnp.float8_e4m3fn),
            pltpu.VMEM((_NB2 * _BR2, _F), jnp.float32),
            pltpu.SMEM((2,), jnp.float32),
        ],
        compiler_params=pltpu.CompilerParams(
            dimension_semantics=("arbitrary", "arbitrary"),
        ),
    )(h1, adjc, Wp, bp)
    return out


# speedup vs baseline: 1.0043x; 1.0043x over previous
import jax
import jax.numpy as jnp
from jax.experimental import pallas as pl
from jax.experimental.pallas import tpu as pltpu

_N = 10000
_F = 128
_BR1 = 400
_NB1 = _N // _BR1
_BR2 = 1000
_NB2 = _N // _BR2


def _layer1_body(xp_ref, adj_ref, W_ref, b_ref, h1_ref, adjc_ref, S_ref):
    i = pl.program_id(0)

    @pl.when(i == 0)
    def _compute_support():
        S_ref[...] = jnp.dot(xp_ref[...], W_ref[0],
                             preferred_element_type=jnp.float32)

    a = adj_ref[...]
    adjc_ref[...] = a.astype(jnp.float8_e4m3fn)
    acc = jnp.dot(a, S_ref[...], preferred_element_type=jnp.float32)
    h1_ref[...] = jnp.maximum(acc + b_ref[0], 0.0)


def _layer23_body(h1_ref, adjc_ref, W_ref, b_ref, out_ref,
                  Sc_ref, H_ref, sc_ref):
    l = pl.program_id(0)
    i = pl.program_id(1)

    def _quantize_support(prev, W):
        S = jnp.dot(prev, W, preferred_element_type=jnp.float32)
        s1 = jnp.maximum(jnp.max(jnp.abs(S)) / 448.0, 1e-30)
        Sh = (S / s1).astype(jnp.float8_e4m3fn)
        Rr = S - Sh.astype(jnp.float32) * s1
        s2 = jnp.maximum(jnp.max(jnp.abs(Rr)) / 448.0, 1e-30)
        Sc_ref[:, 0:_F] = Sh
        Sc_ref[:, _F:2 * _F] = (Rr / s2).astype(jnp.float8_e4m3fn)
        sc_ref[0] = s1
        sc_ref[1] = s2

    @pl.when(jnp.logical_and(l == 0, i == 0))
    def _support_l2():
        _quantize_support(h1_ref[...], W_ref[1])

    @pl.when(jnp.logical_and(l == 1, i == 0))
    def _support_l3():
        _quantize_support(H_ref[...], W_ref[2])

    acc_w = jnp.dot(adjc_ref[...], Sc_ref[...],
                    preferred_element_type=jnp.float32)
    acc = (sc_ref[0] * acc_w[:, 0:_F]
           + sc_ref[1] * acc_w[:, _F:2 * _F]
           + b_ref[l + 1])
    h = jnp.where(l == 0, jnp.maximum(acc, 0.0), acc)
    H_ref[pl.ds(i * _BR2, _BR2), :] = h
    out_ref[...] = h


def kernel(x, adj, W1, b1, W2, b2, W3, b3):
    xp = jnp.zeros((_N, _F), jnp.float32).at[:, :2].set(x)
    Wp = (jnp.zeros((3, _F, _F), jnp.float32)
          .at[0, :2, :5].set(W1)
          .at[1, :5, :5].set(W2)
          .at[2, :5, :2].set(W3))
    bp = (jnp.zeros((3, 1, _F), jnp.float32)
          .at[0, 0, :5].set(b1)
          .at[1, 0, :5].set(b2)
          .at[2, 0, :2].set(b3))

    h1, adjc = pl.pallas_call(
        _layer1_body,
        grid=(_NB1,),
        in_specs=[
            pl.BlockSpec((_N, _F), lambda i: (0, 0)),
            pl.BlockSpec((_BR1, _N), lambda i: (i, 0)),
            pl.BlockSpec((3, _F, _F), lambda i: (0, 0, 0)),
            pl.BlockSpec((3, 1, _F), lambda i: (0, 0, 0)),
        ],
        out_specs=[
            pl.BlockSpec((_BR1, _F), lambda i: (i, 0)),
            pl.BlockSpec((_BR1, _N), lambda i: (i, 0)),
        ],
        out_shape=[
            jax.ShapeDtypeStruct((_N, _F), jnp.float32),
            jax.ShapeDtypeStruct((_N, _N), jnp.float8_e4m3fn),
        ],
        scratch_shapes=[pltpu.VMEM((_N, _F), jnp.float32)],
        compiler_params=pltpu.CompilerParams(
            dimension_semantics=("arbitrary",),
        ),
    )(xp, adj, Wp, bp)

    out = pl.pallas_call(
        _layer23_body,
        grid=(2, _NB2),
        in_specs=[
            pl.BlockSpec((_N, _F), lambda l, i: (0, 0)),
            pl.BlockSpec((_BR2, _N), lambda l, i: (i, 0)),
            pl.BlockSpec((3, _F, _F), lambda l, i: (0, 0, 0)),
            pl.BlockSpec((3, 1, _F), lambda l, i: (0, 0, 0)),
        ],
        out_specs=pl.BlockSpec((_BR2, _F), lambda l, i: (i, 0)),
        out_shape=jax.ShapeDtypeStruct((_N, _F), jnp.float32),
        scratch_shapes=[
            pltpu.VMEM((_N, 2 * _F), jnp.float8_e4m3fn),
            pltpu.VMEM((_N, _F), jnp.float32),
            pltpu.SMEM((2,), jnp.float32),
        ],
        compiler_params=pltpu.CompilerParams(
            dimension_semantics=("arbitrary", "arbitrary"),
        ),
    )(h1, adjc, Wp, bp)
    return out[:, 0:2]
